# baseline (speedup 1.0000x reference)
import jax
import jax.numpy as jnp
from jax import lax
from jax.experimental import pallas as pl
from jax.experimental.pallas import tpu as pltpu
from jax.experimental.pallas import tpu_sc as plsc

NC = 2
NS = 16
NW = NC * NS
L = 16
CHUNK = 128
BN = 2000


def _mesh():
    return plsc.VectorSubcoreMesh(core_axis_name="c", subcore_axis_name="s")


def _deg_call(n_acc, c_chunks):
    seg = n_acc // NS

    def body(dstq, out, dst_v, hist_v, tmp_v, acc_v, slab_sh):
        c = lax.axis_index("c")
        s = lax.axis_index("s")
        wid = c * NS + s
        pltpu.sync_copy(dstq.at[pl.ds(wid * c_chunks, c_chunks)], dst_v)
        zero = jnp.zeros((L,), jnp.float32)

        def zbody(i, carry):
            hist_v[pl.ds(i * L, L)] = zero
            return carry

        lax.fori_loop(0, n_acc // L, zbody, 0)
        ones = jnp.ones((L,), jnp.float32)

        def step(j, carry):
            for k in range(CHUNK // L):
                v = dst_v[j, pl.ds(k * L, L)]
                plsc.addupdate_scatter(hist_v, [v], ones)
            return carry

        lax.fori_loop(0, c_chunks, step, 0)
        pltpu.sync_copy(hist_v, slab_sh.at[s])
        plsc.subcore_barrier()
        base = s * seg
        pltpu.sync_copy(slab_sh.at[0, pl.ds(base, seg)], acc_v)

        def red(t, carry):
            pltpu.sync_copy(slab_sh.at[t, pl.ds(base, seg)], tmp_v)

            def add16(i, carry2):
                acc_v[pl.ds(i * L, L)] = (acc_v[pl.ds(i * L, L)]
                                          + tmp_v[pl.ds(i * L, L)])
                return carry2

            lax.fori_loop(0, seg // L, add16, 0)
            return carry

        lax.fori_loop(1, NS, red, 0)
        pltpu.sync_copy(acc_v, out.at[c, pl.ds(base, seg)])

    return pl.kernel(
        body,
        out_type=jax.ShapeDtypeStruct((NC, n_acc), jnp.float32),
        mesh=_mesh(),
        compiler_params=pltpu.CompilerParams(needs_layout_passes=False),
        scratch_types=[
            pltpu.VMEM((c_chunks, CHUNK), jnp.int32),
            pltpu.VMEM((n_acc,), jnp.float32),
            pltpu.VMEM((seg,), jnp.float32),
            pltpu.VMEM((seg,), jnp.float32),
            pltpu.VMEM_SHARED((NS, n_acc), jnp.float32),
        ],
    )


W = 8


def _prop_call(n_acc, d, c_chunks):
    rpt = n_acc // NS
    nblk = rpt // CHUNK
    gtot = c_chunks // W

    def body(table, srcq, dstq3, zrows, out,
             src_v, dw0, dw1, rows0, rows1, acc_sh, gsem0, gsem1, dsem0,
             dsem1):
        c = lax.axis_index("c")
        s = lax.axis_index("s")
        wid = c * NS + s
        pltpu.sync_copy(srcq.at[pl.ds(wid * c_chunks, c_chunks)], src_v)
        base = s * rpt
        pltpu.sync_copy(zrows, rows0)
        for b in range(nblk):
            pltpu.sync_copy(rows0, acc_sh.at[pl.ds(base + b * CHUNK, CHUNK)])
        plsc.subcore_barrier()
        rows = (rows0, rows1)
        gsems = (gsem0, gsem1)
        dw = (dw0, dw1)
        dsems = (dsem0, dsem1)
        gbase = wid * gtot
        pltpu.sync_copy(dstq3.at[gbase], dw0)
        pltpu.async_copy(dstq3.at[gbase + 1], dw1, dsem1)
        pltpu.async_copy(table.at[src_v.at[0]], rows0, gsem0)
        pltpu.async_copy(table.at[src_v.at[1]], rows1, gsem1)

        def pair(gp, carry):
            for wb in range(2):
                g = gp * 2 + wb

                @pl.when(g > 0)
                def _():
                    pltpu.make_async_copy(dstq3.at[gbase + g], dw[wb],
                                          dsems[wb]).wait()

                for k in range(W):
                    j = g * W + k
                    rb = k % 2
                    pltpu.make_async_copy(table.at[src_v.at[j]], rows[rb],
                                          gsems[rb]).wait()
                    pltpu.sync_copy(rows[rb], acc_sh.at[dw[wb].at[k]],
                                    add=True)

                    @pl.when(j + 2 < c_chunks)
                    def _():
                        pltpu.async_copy(table.at[src_v.at[j + 2]], rows[rb],
                                         gsems[rb])

                @pl.when(g + 2 < gtot)
                def _():
                    pltpu.async_copy(dstq3.at[gbase + g + 2], dw[wb],
                                     dsems[wb])
            return carry

        lax.fori_loop(0, gtot // 2, pair, 0)
        plsc.subcore_barrier()
        pltpu.sync_copy(acc_sh.at[pl.ds(base, rpt)],
                        out.at[c, pl.ds(base, rpt)])

    return pl.kernel(
        body,
        out_type=jax.ShapeDtypeStruct((NC, n_acc, d), jnp.float32),
        mesh=_mesh(),
        scratch_types=[
            pltpu.VMEM((c_chunks, CHUNK), jnp.int32),
            pltpu.VMEM((W, CHUNK), jnp.int32),
            pltpu.VMEM((W, CHUNK), jnp.int32),
            pltpu.VMEM((CHUNK, d), jnp.float32),
            pltpu.VMEM((CHUNK, d), jnp.float32),
            pltpu.VMEM_SHARED((n_acc, d), jnp.float32),
            pltpu.SemaphoreType.DMA,
            pltpu.SemaphoreType.DMA,
            pltpu.SemaphoreType.DMA,
            pltpu.SemaphoreType.DMA,
        ],
    )


def _prep_body(d0_ref, d1_ref, x_ref, dinv_ref, xs_ref):
    deg = d0_ref[...] + d1_ref[...] + 1.0
    dinv = lax.rsqrt(deg)
    dinv_ref[...] = dinv
    xs_ref[...] = x_ref[...] * dinv


def _prep(d0, d1, x):
    n, din = x.shape
    col = lambda i: (i, 0)
    return pl.pallas_call(
        _prep_body,
        grid=(n // BN,),
        in_specs=[pl.BlockSpec((BN, 1), col),
                  pl.BlockSpec((BN, 1), col),
                  pl.BlockSpec((BN, din), col)],
        out_specs=[pl.BlockSpec((BN, 1), col),
                   pl.BlockSpec((BN, din), col)],
        out_shape=[jax.ShapeDtypeStruct((n, 1), jnp.float32),
                   jax.ShapeDtypeStruct((n, din), jnp.float32)],
    )(d0, d1, x)


def _mid_body(p0_ref, p1_ref, xs_ref, dinv_ref, w1_ref, b1_ref, hs_ref):
    dinv = dinv_ref[...]
    agg = dinv * (p0_ref[0] + p1_ref[0] + xs_ref[...])
    h = jnp.dot(agg, w1_ref[...], preferred_element_type=jnp.float32)
    h = jnp.maximum(h + b1_ref[...], 0.0)
    hs_ref[...] = dinv * h


def _mid(p, xs, dinv, w1, b1):
    n, din = xs.shape
    dhid = w1.shape[1]
    col = lambda i: (i, 0)
    zero = lambda i: (0, 0)
    return pl.pallas_call(
        _mid_body,
        grid=(n // BN,),
        in_specs=[pl.BlockSpec((1, BN, din), lambda i: (0, i, 0)),
                  pl.BlockSpec((1, BN, din), lambda i: (1, i, 0)),
                  pl.BlockSpec((BN, din), col),
                  pl.BlockSpec((BN, 1), col),
                  pl.BlockSpec((din, dhid), zero),
                  pl.BlockSpec((1, dhid), zero)],
        out_specs=pl.BlockSpec((BN, dhid), col),
        out_shape=jax.ShapeDtypeStruct((n, dhid), jnp.float32),
    )(p, p, xs, dinv, w1, b1)


def _final_body(q0_ref, q1_ref, hs_ref, dinv_ref, w2_ref, b2_ref, out_ref):
    agg = dinv_ref[...] * (q0_ref[0] + q1_ref[0] + hs_ref[...])
    out_ref[...] = (jnp.dot(agg, w2_ref[...],
                            preferred_element_type=jnp.float32) + b2_ref[...])


def _final(q, hs, dinv, w2, b2):
    n, dhid = hs.shape
    dout = w2.shape[1]
    col = lambda i: (i, 0)
    zero = lambda i: (0, 0)
    return pl.pallas_call(
        _final_body,
        grid=(n // BN,),
        in_specs=[pl.BlockSpec((1, BN, dhid), lambda i: (0, i, 0)),
                  pl.BlockSpec((1, BN, dhid), lambda i: (1, i, 0)),
                  pl.BlockSpec((BN, dhid), col),
                  pl.BlockSpec((BN, 1), col),
                  pl.BlockSpec((dhid, dout), zero),
                  pl.BlockSpec((1, dout), zero)],
        out_specs=pl.BlockSpec((BN, dout), col),
        out_shape=jax.ShapeDtypeStruct((n, dout), jnp.float32),
    )(q, q, hs, dinv, w2, b2)


def kernel(x, edge_index, W1, b1, W2, b2):
    x = x.astype(jnp.float32)
    n, din = x.shape
    e = edge_index.shape[1]
    dout = W2.shape[1]
    src = edge_index[0].astype(jnp.int32)
    dst = edge_index[1].astype(jnp.int32)

    c_chunks = -(-(-(-e // (NW * CHUNK))) // (2 * W)) * (2 * W)
    e_pad = NW * c_chunks * CHUNK
    n_acc = -(-(n + 1) // (NS * CHUNK)) * NS * CHUNK
    pad = e_pad - e
    pad_src = jnp.arange(pad, dtype=jnp.int32) % n
    pad_dst = n + jnp.arange(pad, dtype=jnp.int32) % (n_acc - n)
    srcp = jnp.concatenate([src, pad_src])
    dstp = jnp.concatenate([dst, pad_dst])

    degp = _deg_call(n_acc, c_chunks)(dstp.reshape(-1, CHUNK))
    d0 = degp[0, :n].reshape(n, 1)
    d1 = degp[1, :n].reshape(n, 1)
    dinv, xs = _prep(d0, d1, x)

    srcq = srcp.reshape(-1, CHUNK)
    dstq3 = dstp.reshape(-1, W, CHUNK)
    z_hid = jnp.zeros((CHUNK, din), jnp.float32)
    prop = _prop_call(n_acc, din, c_chunks)
    p = prop(xs, srcq, dstq3, z_hid)
    hs = _mid(p, xs, dinv, W1, b1.reshape(1, -1))

    q = prop(hs, srcq, dstq3, z_hid)
    return _final(q, hs, dinv, W2, b2.reshape(1, -1))


# speedup vs baseline: 1.0013x; 1.0013x over previous
import jax
import jax.numpy as jnp
from jax import lax
from jax.experimental import pallas as pl
from jax.experimental.pallas import tpu as pltpu
from jax.experimental.pallas import tpu_sc as plsc

NC = 2
NS = 16
NW = NC * NS
L = 16
CHUNK = 128
BN = 2000


def _mesh():
    return plsc.VectorSubcoreMesh(core_axis_name="c", subcore_axis_name="s")


def _deg_call(n_acc, c_chunks):
    seg = n_acc // NS

    def body(dstq, out, dst_v, hist_v, tmp_v, acc_v, slab_sh):
        c = lax.axis_index("c")
        s = lax.axis_index("s")
        wid = c * NS + s
        pltpu.sync_copy(dstq.at[pl.ds(wid * c_chunks, c_chunks)], dst_v)
        zero = jnp.zeros((L,), jnp.float32)

        def zbody(i, carry):
            hist_v[pl.ds(i * L, L)] = zero
            return carry

        lax.fori_loop(0, n_acc // L, zbody, 0)
        ones = jnp.ones((L,), jnp.float32)

        def step(j, carry):
            for k in range(CHUNK // L):
                v = dst_v[j, pl.ds(k * L, L)]
                plsc.addupdate_scatter(hist_v, [v], ones)
            return carry

        lax.fori_loop(0, c_chunks, step, 0)
        pltpu.sync_copy(hist_v, slab_sh.at[s])
        plsc.subcore_barrier()
        base = s * seg
        pltpu.sync_copy(slab_sh.at[0, pl.ds(base, seg)], acc_v)

        def red(t, carry):
            pltpu.sync_copy(slab_sh.at[t, pl.ds(base, seg)], tmp_v)

            def add16(i, carry2):
                acc_v[pl.ds(i * L, L)] = (acc_v[pl.ds(i * L, L)]
                                          + tmp_v[pl.ds(i * L, L)])
                return carry2

            lax.fori_loop(0, seg // L, add16, 0)
            return carry

        lax.fori_loop(1, NS, red, 0)
        pltpu.sync_copy(acc_v, out.at[c, pl.ds(base, seg)])

    return pl.kernel(
        body,
        out_type=jax.ShapeDtypeStruct((NC, n_acc), jnp.float32),
        mesh=_mesh(),
        compiler_params=pltpu.CompilerParams(needs_layout_passes=False),
        scratch_types=[
            pltpu.VMEM((c_chunks, CHUNK), jnp.int32),
            pltpu.VMEM((n_acc,), jnp.float32),
            pltpu.VMEM((seg,), jnp.float32),
            pltpu.VMEM((seg,), jnp.float32),
            pltpu.VMEM_SHARED((NS, n_acc), jnp.float32),
        ],
    )


W = 8


def _prop_call(n_acc, d, c_chunks):
    rpt = n_acc // NS
    nblk = rpt // CHUNK
    gtot = c_chunks // W

    def body(table, srcq, dstq3, zrows, out,
             src_v, dw0, dw1, rows0, rows1, acc_sh, gsem0, gsem1, dsem0,
             dsem1):
        c = lax.axis_index("c")
        s = lax.axis_index("s")
        wid = c * NS + s
        pltpu.sync_copy(srcq.at[pl.ds(wid * c_chunks, c_chunks)], src_v)
        base = s * rpt
        pltpu.sync_copy(zrows, rows0)
        for b in range(nblk):
            pltpu.sync_copy(rows0, acc_sh.at[pl.ds(base + b * CHUNK, CHUNK)])
        plsc.subcore_barrier()
        rows = (rows0, rows1)
        gsems = (gsem0, gsem1)
        dw = (dw0, dw1)
        dsems = (dsem0, dsem1)
        gbase = wid * gtot
        pltpu.sync_copy(dstq3.at[gbase], dw0)
        pltpu.async_copy(dstq3.at[gbase + 1], dw1, dsem1)
        pltpu.async_copy(table.at[src_v.at[0]], rows0, gsem0)
        pltpu.async_copy(table.at[src_v.at[1]], rows1, gsem1)

        def pair(gp, carry):
            for wb in range(2):
                g = gp * 2 + wb

                @pl.when(g > 0)
                def _():
                    pltpu.make_async_copy(dstq3.at[gbase + g], dw[wb],
                                          dsems[wb]).wait()

                for k in range(W):
                    j = g * W + k
                    rb = k % 2
                    pltpu.make_async_copy(table.at[src_v.at[j]], rows[rb],
                                          gsems[rb]).wait()
                    pltpu.sync_copy(rows[rb], acc_sh.at[dw[wb].at[k]],
                                    add=True)

                    @pl.when(j + 2 < c_chunks)
                    def _():
                        pltpu.async_copy(table.at[src_v.at[j + 2]], rows[rb],
                                         gsems[rb])

                @pl.when(g + 2 < gtot)
                def _():
                    pltpu.async_copy(dstq3.at[gbase + g + 2], dw[wb],
                                     dsems[wb])
            return carry

        lax.fori_loop(0, gtot // 2, pair, 0)
        plsc.subcore_barrier()
        pltpu.sync_copy(acc_sh.at[pl.ds(base, rpt)],
                        out.at[c, pl.ds(base, rpt)])

    return pl.kernel(
        body,
        out_type=jax.ShapeDtypeStruct((NC, n_acc, d), jnp.float32),
        mesh=_mesh(),
        scratch_types=[
            pltpu.VMEM((c_chunks, CHUNK), jnp.int32),
            pltpu.VMEM((W, CHUNK), jnp.int32),
            pltpu.VMEM((W, CHUNK), jnp.int32),
            pltpu.VMEM((CHUNK, d), jnp.float32),
            pltpu.VMEM((CHUNK, d), jnp.float32),
            pltpu.VMEM_SHARED((n_acc, d), jnp.float32),
            pltpu.SemaphoreType.DMA,
            pltpu.SemaphoreType.DMA,
            pltpu.SemaphoreType.DMA,
            pltpu.SemaphoreType.DMA,
        ],
    )


def _prep_body(d0_ref, d1_ref, x_ref, dinv_ref, xs_ref):
    deg = d0_ref[...] + d1_ref[...] + 1.0
    dinv = lax.rsqrt(deg)
    dinv_ref[...] = dinv
    xs_ref[...] = x_ref[...] * dinv


def _prep(d0, d1, x):
    n, din = x.shape
    col = lambda i: (i, 0)
    return pl.pallas_call(
        _prep_body,
        grid=(n // BN,),
        in_specs=[pl.BlockSpec((BN, 1), col),
                  pl.BlockSpec((BN, 1), col),
                  pl.BlockSpec((BN, din), col)],
        out_specs=[pl.BlockSpec((BN, 1), col),
                   pl.BlockSpec((BN, din), col)],
        out_shape=[jax.ShapeDtypeStruct((n, 1), jnp.float32),
                   jax.ShapeDtypeStruct((n, din), jnp.float32)],
    )(d0, d1, x)


def _mid_body(p0_ref, p1_ref, xs_ref, dinv_ref, w1_ref, b1_ref, hs_ref):
    dinv = dinv_ref[...]
    agg = dinv * (p0_ref[0] + p1_ref[0] + xs_ref[...])
    h = jnp.dot(agg, w1_ref[...], preferred_element_type=jnp.float32)
    h = jnp.maximum(h + b1_ref[...], 0.0)
    hs_ref[...] = dinv * h


def _mid(p, xs, dinv, w1, b1):
    n, din = xs.shape
    dhid = w1.shape[1]
    col = lambda i: (i, 0)
    zero = lambda i: (0, 0)
    return pl.pallas_call(
        _mid_body,
        grid=(n // BN,),
        in_specs=[pl.BlockSpec((1, BN, din), lambda i: (0, i, 0)),
                  pl.BlockSpec((1, BN, din), lambda i: (1, i, 0)),
                  pl.BlockSpec((BN, din), col),
                  pl.BlockSpec((BN, 1), col),
                  pl.BlockSpec((din, dhid), zero),
                  pl.BlockSpec((1, dhid), zero)],
        out_specs=pl.BlockSpec((BN, dhid), col),
        out_shape=jax.ShapeDtypeStruct((n, dhid), jnp.float32),
    )(p, p, xs, dinv, w1, b1)


def _final_body(q0_ref, q1_ref, hs_ref, dinv_ref, w2_ref, b2_ref, out_ref):
    agg = dinv_ref[...] * (q0_ref[0] + q1_ref[0] + hs_ref[...])
    out_ref[...] = (jnp.dot(agg, w2_ref[...],
                            preferred_element_type=jnp.float32) + b2_ref[...])


def _final(q, hs, dinv, w2, b2):
    n, dhid = hs.shape
    dout = w2.shape[1]
    col = lambda i: (i, 0)
    zero = lambda i: (0, 0)
    return pl.pallas_call(
        _final_body,
        grid=(n // BN,),
        in_specs=[pl.BlockSpec((1, BN, dhid), lambda i: (0, i, 0)),
                  pl.BlockSpec((1, BN, dhid), lambda i: (1, i, 0)),
                  pl.BlockSpec((BN, dhid), col),
                  pl.BlockSpec((BN, 1), col),
                  pl.BlockSpec((dhid, dout), zero),
                  pl.BlockSpec((1, dout), zero)],
        out_specs=pl.BlockSpec((BN, dout), col),
        out_shape=jax.ShapeDtypeStruct((n, dout), jnp.float32),
    )(q, q, hs, dinv, w2, b2)


def kernel(x, edge_index, W1, b1, W2, b2):
    x = x.astype(jnp.float32)
    n, din = x.shape
    e = edge_index.shape[1]
    src = edge_index[0].astype(jnp.int32)
    dst = edge_index[1].astype(jnp.int32)

    c_chunks = -(-(-(-e // (NW * CHUNK))) // (2 * W)) * (2 * W)
    e_pad = NW * c_chunks * CHUNK
    n_acc = -(-(n + 1) // (NS * CHUNK)) * NS * CHUNK
    pad = e_pad - e
    pad_src = jnp.arange(pad, dtype=jnp.int32) % n
    pad_dst = n + jnp.arange(pad, dtype=jnp.int32) % (n_acc - n)
    srcp = jnp.concatenate([src, pad_src])
    dstp = jnp.concatenate([dst, pad_dst])

    degp = _deg_call(n_acc, c_chunks)(dstp.reshape(-1, CHUNK))
    d0 = degp[0, :n].reshape(n, 1)
    d1 = degp[1, :n].reshape(n, 1)
    dinv, xs = _prep(d0, d1, x)

    srcq = srcp.reshape(-1, CHUNK)
    dstq3 = dstp.reshape(-1, W, CHUNK)
    z_hid = jnp.zeros((CHUNK, din), jnp.float32)
    prop = _prop_call(n_acc, din, c_chunks)
    p = prop(xs, srcq, dstq3, z_hid)
    hs = _mid(p, xs, dinv, W1, b1.reshape(1, -1))

    q = prop(hs, srcq, dstq3, z_hid)
    return _final(q, hs, dinv, W2, b2.reshape(1, -1))
